# baseline (speedup 1.0000x reference)
import functools

import jax
import jax.numpy as jnp
from jax import lax
from jax.experimental import pallas as pl
from jax.experimental.pallas import tpu as pltpu
from jax.experimental.pallas import tpu_sc as plsc

NBRICK = 100000
DIM = 64
BATCH = 16384

_NC = 2
_NS = 16
_L = 16
_NW = _NC * _NS
_DPW = DIM // _NW
_OCHUNK = 4096
_NOCHUNK = BATCH // _OCHUNK
_VPC = _OCHUNK // _L
_UNROLL = 64

_mesh = plsc.VectorSubcoreMesh(core_axis_name="c", subcore_axis_name="s")


@functools.partial(
    pl.kernel,
    mesh=_mesh,
    out_type=jax.ShapeDtypeStruct((DIM, BATCH), jnp.float32),
    scratch_types=[
        pltpu.VMEM((BATCH,), jnp.int32),
        pltpu.VMEM((NBRICK,), jnp.float32),
        pltpu.VMEM((2, _OCHUNK), jnp.float32),
        pltpu.SemaphoreType.DMA,
        pltpu.SemaphoreType.DMA,
        pltpu.SemaphoreType.DMA,
    ],
    compiler_params=pltpu.CompilerParams(
        use_tc_tiling_on_sc=True, needs_layout_passes=False
    ),
)
def _embed_t(x1_hbm, embt_hbm, outt_hbm, idx_v, row_v, ob, rsem, osem, xsem):
    wid = lax.axis_index("s") * _NC + lax.axis_index("c")
    d0 = wid * _DPW

    x_copy = pltpu.async_copy(x1_hbm, idx_v, xsem)
    row_copy = pltpu.async_copy(embt_hbm.at[d0], row_v, rsem)
    x_copy.wait()

    rcp = jnp.float32(1.0 / 90.0)

    @plsc.parallel_loop(0, BATCH // _L, unroll=_UNROLL)
    def _div_body(k):
        off = k * _L
        v = idx_v[pl.ds(off, _L)]
        q = (v.astype(jnp.float32) * rcp).astype(jnp.int32)
        r = v - q * 90
        idx_v[pl.ds(off, _L)] = lax.select(r >= 90, q + 1, q)

    row_copy.wait()

    for p in range(_DPW):
        d = d0 + p
        if p > 0:
            pltpu.sync_copy(embt_hbm.at[d], row_v)
        outs = []
        for c in range(_NOCHUNK):
            bsel = c % 2
            if len(outs) >= 2:
                outs[-2].wait()

            @plsc.parallel_loop(0, _VPC, unroll=_UNROLL)
            def _gather_body(k, c=c, bsel=bsel):
                rel = k * _L
                iv = idx_v[pl.ds(c * _OCHUNK + rel, _L)]
                ob[bsel, pl.ds(rel, _L)] = plsc.load_gather(row_v, [iv])
            outs.append(
                pltpu.async_copy(
                    ob.at[bsel],
                    outt_hbm.at[d, pl.ds(c * _OCHUNK, _OCHUNK)],
                    osem,
                )
            )
        for o in outs[-2:]:
            o.wait()


def kernel(x, emb):
    x1 = x[:, 1].astype(jnp.int32)
    out_t = _embed_t(x1, emb.T)
    return out_t.T


# speedup vs baseline: 1.3238x; 1.3238x over previous
import functools

import jax
import jax.numpy as jnp
from jax import lax
from jax.experimental import pallas as pl
from jax.experimental.pallas import tpu as pltpu
from jax.experimental.pallas import tpu_sc as plsc

NBRICK = 100000
DIM = 64
BATCH = 16384

_NC = 2
_NS = 16
_L = 16
_NW = _NC * _NS
_DPW = DIM // _NW
_OCHUNK = 4096
_NOCHUNK = BATCH // _OCHUNK
_VPC = _OCHUNK // _L
_UNROLL = 16

_mesh = plsc.VectorSubcoreMesh(core_axis_name="c", subcore_axis_name="s")


@functools.partial(
    pl.kernel,
    mesh=_mesh,
    out_type=jax.ShapeDtypeStruct((DIM, BATCH), jnp.float32),
    scratch_types=[
        pltpu.VMEM((BATCH,), jnp.int32),
        pltpu.VMEM((NBRICK,), jnp.float32),
        pltpu.VMEM((2, _OCHUNK), jnp.float32),
        pltpu.SemaphoreType.DMA,
        pltpu.SemaphoreType.DMA,
        pltpu.SemaphoreType.DMA,
    ],
    compiler_params=pltpu.CompilerParams(
        use_tc_tiling_on_sc=True, needs_layout_passes=False
    ),
)
def _embed_t(x1_hbm, embt_hbm, outt_hbm, idx_v, row_v, ob, rsem, osem, xsem):
    wid = lax.axis_index("s") * _NC + lax.axis_index("c")
    d0 = wid * _DPW

    x_copy = pltpu.async_copy(x1_hbm, idx_v, xsem)
    row_copy = pltpu.async_copy(embt_hbm.at[d0], row_v, rsem)
    x_copy.wait()

    rcp = jnp.float32(1.0 / 90.0)

    @plsc.parallel_loop(0, BATCH // _L, unroll=_UNROLL)
    def _div_body(k):
        off = k * _L
        v = idx_v[pl.ds(off, _L)]
        q = (v.astype(jnp.float32) * rcp).astype(jnp.int32)
        r = v - q * 90
        idx_v[pl.ds(off, _L)] = lax.select(r >= 90, q + 1, q)

    row_copy.wait()

    for p in range(_DPW):
        d = d0 + p
        if p > 0:
            pltpu.sync_copy(embt_hbm.at[d], row_v)
        outs = []
        for c in range(_NOCHUNK):
            bsel = c % 2
            if len(outs) >= 2:
                outs[-2].wait()

            @plsc.parallel_loop(0, _VPC, unroll=_UNROLL)
            def _gather_body(k, c=c, bsel=bsel):
                rel = k * _L
                iv = idx_v[pl.ds(c * _OCHUNK + rel, _L)]
                ob[bsel, pl.ds(rel, _L)] = plsc.load_gather(row_v, [iv])
            outs.append(
                pltpu.async_copy(
                    ob.at[bsel],
                    outt_hbm.at[d, pl.ds(c * _OCHUNK, _OCHUNK)],
                    osem,
                )
            )
        for o in outs[-2:]:
            o.wait()


def kernel(x, emb):
    x1 = x[:, 1].astype(jnp.int32)
    out_t = _embed_t(x1, emb.T)
    return out_t.T


# speedup vs baseline: 1.3506x; 1.0203x over previous
import functools

import jax
import jax.numpy as jnp
from jax import lax
from jax.experimental import pallas as pl
from jax.experimental.pallas import tpu as pltpu
from jax.experimental.pallas import tpu_sc as plsc

NBRICK = 100000
DIM = 64
BATCH = 16384

_NC = 2
_NS = 16
_L = 16
_NW = _NC * _NS
_DPW = DIM // _NW
_OCHUNK = 4096
_NOCHUNK = BATCH // _OCHUNK
_VPC = _OCHUNK // _L
_UNROLL = 8

_mesh = plsc.VectorSubcoreMesh(core_axis_name="c", subcore_axis_name="s")


@functools.partial(
    pl.kernel,
    mesh=_mesh,
    out_type=jax.ShapeDtypeStruct((DIM, BATCH), jnp.float32),
    scratch_types=[
        pltpu.VMEM((BATCH,), jnp.int32),
        pltpu.VMEM((NBRICK,), jnp.float32),
        pltpu.VMEM((2, _OCHUNK), jnp.float32),
        pltpu.SemaphoreType.DMA,
        pltpu.SemaphoreType.DMA,
        pltpu.SemaphoreType.DMA,
    ],
    compiler_params=pltpu.CompilerParams(
        use_tc_tiling_on_sc=True, needs_layout_passes=False
    ),
)
def _embed_t(x1_hbm, embt_hbm, outt_hbm, idx_v, row_v, ob, rsem, osem, xsem):
    wid = lax.axis_index("s") * _NC + lax.axis_index("c")
    d0 = wid * _DPW

    x_copy = pltpu.async_copy(x1_hbm, idx_v, xsem)
    row_copy = pltpu.async_copy(embt_hbm.at[d0], row_v, rsem)
    x_copy.wait()

    rcp = jnp.float32(1.0 / 90.0)

    @plsc.parallel_loop(0, BATCH // _L, unroll=_UNROLL)
    def _div_body(k):
        off = k * _L
        v = idx_v[pl.ds(off, _L)]
        q = (v.astype(jnp.float32) * rcp).astype(jnp.int32)
        r = v - q * 90
        idx_v[pl.ds(off, _L)] = lax.select(r >= 90, q + 1, q)

    row_copy.wait()

    for p in range(_DPW):
        d = d0 + p
        if p > 0:
            pltpu.sync_copy(embt_hbm.at[d], row_v)
        outs = []
        for c in range(_NOCHUNK):
            bsel = c % 2
            if len(outs) >= 2:
                outs[-2].wait()

            @plsc.parallel_loop(0, _VPC, unroll=_UNROLL)
            def _gather_body(k, c=c, bsel=bsel):
                rel = k * _L
                iv = idx_v[pl.ds(c * _OCHUNK + rel, _L)]
                ob[bsel, pl.ds(rel, _L)] = plsc.load_gather(row_v, [iv])
            outs.append(
                pltpu.async_copy(
                    ob.at[bsel],
                    outt_hbm.at[d, pl.ds(c * _OCHUNK, _OCHUNK)],
                    osem,
                )
            )
        for o in outs[-2:]:
            o.wait()


def kernel(x, emb):
    x1 = x[:, 1].astype(jnp.int32)
    out_t = _embed_t(x1, emb.T)
    return out_t.T
